# baseline (speedup 1.0000x reference)
import functools

import jax
import jax.numpy as jnp
from jax.experimental import pallas as pl
from jax.experimental.pallas import tpu as pltpu
from jax.experimental.pallas import tpu_sc as plsc

N_DIM_EMB = 32
N_ARY_OUT = 64

_GATHER_WINDOW = 128
_BQ = 16384
_BB = 4096
_NCHUNK = 5


def _tc_project_table(tableT, W, b2d):
    v = tableT.shape[1]
    nblk = (v + 2 * _BQ - 1) // (2 * _BQ)

    def body(t_ref, w_ref, b_ref, o_ref):
        res = jax.lax.dot_general(
            t_ref[...], w_ref[...],
            dimension_numbers=(((0,), (0,)), ((), ())),
            preferred_element_type=jnp.float32,
        ) + b_ref[...]
        o_ref[...] = jnp.concatenate([res[:_BQ], res[_BQ:]], axis=1)

    return pl.pallas_call(
        body,
        grid=(nblk,),
        in_specs=[
            pl.BlockSpec((N_DIM_EMB, 2 * _BQ), lambda i: (0, i)),
            pl.BlockSpec((N_DIM_EMB, N_ARY_OUT), lambda i: (0, 0)),
            pl.BlockSpec((1, N_ARY_OUT), lambda i: (0, 0)),
        ],
        out_specs=pl.BlockSpec((_BQ, 2 * N_ARY_OUT), lambda i: (i, 0)),
        out_shape=jax.ShapeDtypeStruct((nblk * _BQ, 2 * N_ARY_OUT), jnp.float32),
    )(tableT, W, b2d)


def _sc_gather(tw2, idx_chunk):
    m = idx_chunk.shape[1]
    mesh = plsc.VectorSubcoreMesh(core_axis_name="core", subcore_axis_name="subcore")

    @functools.partial(
        pl.kernel,
        out_type=jax.ShapeDtypeStruct((m, 2 * N_ARY_OUT), jnp.float32),
        mesh=mesh,
    )
    def gather_kernel(tw_hbm, idx_hbm, out_hbm):
        def body(i_vmem, o_vmem):
            pltpu.sync_copy(tw_hbm.at[i_vmem.at[0]], o_vmem)

        pltpu.emit_pipeline(
            body,
            grid=(m // _GATHER_WINDOW,),
            in_specs=[pl.BlockSpec((1, _GATHER_WINDOW), lambda i: (0, i))],
            out_specs=[pl.BlockSpec((_GATHER_WINDOW, 2 * N_ARY_OUT), lambda i: (i, 0))],
            core_axis_name=("core", "subcore"),
            dimension_semantics=(pltpu.PARALLEL,),
        )(idx_hbm, out_hbm)

    return gather_kernel(tw2, idx_chunk)


def _tc_select_transpose_chunk(out3_in, g2s, paritys, s, dchunk, n_digits, batch):
    nj = batch // _BB

    def body(g_ref, p_ref, o_ref):
        gt = jnp.transpose(g_ref[...], (1, 0))
        par = p_ref[0]
        sel = jnp.where(par == 0, gt[:N_ARY_OUT, :], gt[N_ARY_OUT:, :])
        o_ref[...] = sel[None]

    def body_aliased(o_in_ref, g_ref, p_ref, o_ref):
        body(g_ref, p_ref, o_ref)

    data_specs = [
        pl.BlockSpec((_BB, 2 * N_ARY_OUT), lambda d, j: (d * nj + j, 0)),
        pl.BlockSpec((1, 1, _BB), lambda d, j: (d, 0, j)),
    ]
    out_spec = pl.BlockSpec(
        (1, N_ARY_OUT, _BB), lambda d, j: (s * dchunk + d, 0, j))
    out_shape = jax.ShapeDtypeStruct((n_digits, N_ARY_OUT, batch), jnp.float32)

    if out3_in is None:
        return pl.pallas_call(
            body,
            grid=(dchunk, nj),
            in_specs=data_specs,
            out_specs=out_spec,
            out_shape=out_shape,
        )(g2s, paritys)
    return pl.pallas_call(
        body_aliased,
        grid=(dchunk, nj),
        in_specs=[pl.BlockSpec(memory_space=pltpu.MemorySpace.HBM)] + data_specs,
        out_specs=out_spec,
        out_shape=out_shape,
        input_output_aliases={0: 0},
    )(out3_in, g2s, paritys)


def kernel(indices, table, W, b):
    batch, n_digits = indices.shape
    m = batch * n_digits
    dchunk = n_digits // _NCHUNK
    mchunk = m // _NCHUNK

    tableT = table.T
    tw2 = _tc_project_table(tableT, W, b.reshape(1, N_ARY_OUT))

    idxT = indices.T
    idxq = ((idxT // (2 * _BQ)) * _BQ + (idxT % _BQ)).reshape(1, m)
    parityT = ((idxT // _BQ) & 1).reshape(n_digits, 1, batch)

    out3 = None
    for s in range(_NCHUNK):
        g2s = _sc_gather(tw2, idxq[:, s * mchunk:(s + 1) * mchunk])
        out3 = _tc_select_transpose_chunk(
            out3, g2s, parityT[s * dchunk:(s + 1) * dchunk],
            s, dchunk, n_digits, batch)
    return out3.transpose(2, 0, 1)
